# baseline (speedup 1.0000x reference)
import functools

import jax
import jax.numpy as jnp
from jax import lax
from jax.experimental import pallas as pl
from jax.experimental.pallas import tpu as pltpu
from jax.experimental.pallas import tpu_sc as plsc

_info = plsc.get_sparse_core_info()
_NC, _NS = _info.num_cores, _info.num_subcores
_NW = _NC * _NS

_CHUNK_ROWS = 16


@functools.partial(jax.jit, static_argnums=(0, 1))
def _copy_rows(seq_len, hidden, emb_table):
    rows_per_w = seq_len // _NW
    nch = rows_per_w // _CHUNK_ROWS
    mesh = plsc.VectorSubcoreMesh(core_axis_name="c", subcore_axis_name="s")

    @functools.partial(
        pl.kernel,
        mesh=mesh,
        out_type=jax.ShapeDtypeStruct((seq_len, hidden), jnp.float32),
        scratch_types=[
            pltpu.VMEM_SHARED((_NS, _CHUNK_ROWS, hidden), jnp.float32),
            pltpu.VMEM_SHARED((_NS, _CHUNK_ROWS, hidden), jnp.float32),
            pltpu.SemaphoreType.DMA,
            pltpu.SemaphoreType.DMA,
            pltpu.SemaphoreType.DMA,
            pltpu.SemaphoreType.DMA,
        ],
    )
    def k(table_hbm, out_hbm, buf0, buf1, si0, si1, so0, so1):
        sid = lax.axis_index("s")
        wid = sid * _NC + lax.axis_index("c")
        base = wid * rows_per_w
        bufs = (buf0, buf1)
        sin = (si0, si1)
        sout = (so0, so1)
        in_h = {}
        out_h = {}

        def start_in(c):
            b = c % 2
            in_h[c] = pltpu.async_copy(
                table_hbm.at[pl.ds(base + c * _CHUNK_ROWS, _CHUNK_ROWS)],
                bufs[b].at[sid],
                sin[b],
            )

        def start_out(c):
            b = c % 2
            out_h[c] = pltpu.async_copy(
                bufs[b].at[sid],
                out_hbm.at[pl.ds(base + c * _CHUNK_ROWS, _CHUNK_ROWS)],
                sout[b],
            )

        start_in(0)
        if nch > 1:
            start_in(1)
        for c in range(nch):
            in_h[c].wait()
            start_out(c)
            if c + 2 < nch:
                out_h[c].wait()
                start_in(c + 2)
        for c in range(max(0, nch - 2), nch):
            out_h[c].wait()

    return k(emb_table)


def kernel(x, emb_table):
    seq_len = x.shape[1]
    hidden = emb_table.shape[1]
    out = _copy_rows(seq_len, hidden, emb_table)
    return out[None]


# speedup vs baseline: 1.0062x; 1.0062x over previous
import functools

import jax
import jax.numpy as jnp
from jax import lax
from jax.experimental import pallas as pl
from jax.experimental.pallas import tpu as pltpu
from jax.experimental.pallas import tpu_sc as plsc

_info = plsc.get_sparse_core_info()
_NC, _NS = _info.num_cores, _info.num_subcores
_NW = _NC * _NS

_CHUNK_ROWS = 16


@functools.partial(jax.jit, static_argnums=(0, 1))
def _copy_rows(seq_len, hidden, emb_table):
    rows_per_w = seq_len // _NW
    nch = rows_per_w // _CHUNK_ROWS
    mesh = plsc.VectorSubcoreMesh(core_axis_name="c", subcore_axis_name="s")

    @functools.partial(
        pl.kernel,
        mesh=mesh,
        out_type=jax.ShapeDtypeStruct((seq_len, hidden), jnp.float32),
        scratch_types=[
            pltpu.VMEM((_CHUNK_ROWS, hidden), jnp.float32),
            pltpu.VMEM((_CHUNK_ROWS, hidden), jnp.float32),
            pltpu.VMEM_SHARED((_NS, _CHUNK_ROWS, hidden), jnp.float32),
            pltpu.VMEM_SHARED((_NS, _CHUNK_ROWS, hidden), jnp.float32),
            pltpu.SemaphoreType.DMA,
            pltpu.SemaphoreType.DMA,
            pltpu.SemaphoreType.DMA,
            pltpu.SemaphoreType.DMA,
        ],
    )
    def k(table_hbm, out_hbm, tb0, tb1, sb0, sb1, si0, si1, so0, so1):
        sid = lax.axis_index("s")
        wid = sid * _NC + lax.axis_index("c")
        base = wid * rows_per_w
        sin = (si0, si1)
        sout = (so0, so1)

        def pipeline(bufs):
            in_h = {}
            out_h = {}

            def start_in(c):
                b = c % 2
                in_h[c] = pltpu.async_copy(
                    table_hbm.at[pl.ds(base + c * _CHUNK_ROWS, _CHUNK_ROWS)],
                    bufs[b],
                    sin[b],
                )

            def start_out(c):
                b = c % 2
                out_h[c] = pltpu.async_copy(
                    bufs[b],
                    out_hbm.at[pl.ds(base + c * _CHUNK_ROWS, _CHUNK_ROWS)],
                    sout[b],
                )

            start_in(0)
            if nch > 1:
                start_in(1)
            for c in range(nch):
                in_h[c].wait()
                start_out(c)
                if c + 2 < nch:
                    out_h[c].wait()
                    start_in(c + 2)
            for c in range(max(0, nch - 2), nch):
                out_h[c].wait()

        @pl.when(sid % 2 == 0)
        def _():
            pipeline((tb0, tb1))

        @pl.when(sid % 2 == 1)
        def _():
            pipeline((sb0.at[sid], sb1.at[sid]))

    return k(emb_table)


def kernel(x, emb_table):
    seq_len = x.shape[1]
    hidden = emb_table.shape[1]
    out = _copy_rows(seq_len, hidden, emb_table)
    return out[None]


# speedup vs baseline: 1.0505x; 1.0440x over previous
import functools

import jax
import jax.numpy as jnp
from jax import lax
from jax.experimental import pallas as pl
from jax.experimental.pallas import tpu as pltpu
from jax.experimental.pallas import tpu_sc as plsc

_info = plsc.get_sparse_core_info()
_NC, _NS = _info.num_cores, _info.num_subcores
_NW = _NC * _NS

_CHUNK_ROWS = 32


@functools.partial(jax.jit, static_argnums=(0, 1))
def _copy_rows(seq_len, hidden, emb_table):
    rows_per_w = seq_len // _NW
    nch = rows_per_w // _CHUNK_ROWS
    mesh = plsc.VectorSubcoreMesh(core_axis_name="c", subcore_axis_name="s")

    @functools.partial(
        pl.kernel,
        mesh=mesh,
        out_type=jax.ShapeDtypeStruct((seq_len, hidden), jnp.float32),
        scratch_types=[
            pltpu.VMEM((_CHUNK_ROWS, hidden), jnp.float32),
            pltpu.VMEM_SHARED((_NS, _CHUNK_ROWS, hidden), jnp.float32),
            pltpu.SemaphoreType.DMA,
            pltpu.SemaphoreType.DMA,
            pltpu.SemaphoreType.DMA,
            pltpu.SemaphoreType.DMA,
        ],
    )
    def k(table_hbm, out_hbm, tb, sb, si0, si1, so0, so1):
        sid = lax.axis_index("s")
        wid = sid * _NC + lax.axis_index("c")
        base = wid * rows_per_w
        bufs = (tb, sb.at[sid])
        sin = (si0, si1)
        sout = (so0, so1)
        in_h = {}
        out_h = {}

        def start_in(c):
            b = c % 2
            in_h[c] = pltpu.async_copy(
                table_hbm.at[pl.ds(base + c * _CHUNK_ROWS, _CHUNK_ROWS)],
                bufs[b],
                sin[b],
            )

        def start_out(c):
            b = c % 2
            out_h[c] = pltpu.async_copy(
                bufs[b],
                out_hbm.at[pl.ds(base + c * _CHUNK_ROWS, _CHUNK_ROWS)],
                sout[b],
            )

        start_in(0)
        if nch > 1:
            start_in(1)
        for c in range(nch):
            in_h[c].wait()
            start_out(c)
            if c + 2 < nch:
                out_h[c].wait()
                start_in(c + 2)
        for c in range(max(0, nch - 2), nch):
            out_h[c].wait()

    return k(emb_table)


def kernel(x, emb_table):
    seq_len = x.shape[1]
    hidden = emb_table.shape[1]
    out = _copy_rows(seq_len, hidden, emb_table)
    return out[None]
